# baseline (speedup 1.0000x reference)
import math

import jax
import jax.numpy as jnp
from jax import lax
from jax.experimental import pallas as pl
from jax.experimental.pallas import tpu as pltpu
from jax.experimental.pallas import tpu_sc as plsc

VOCAB = 1000000
D = 128
BATCH = 4096
SEQ = 200

NC = 2
NS = 16
NW = NC * NS
B = BATCH * SEQ
B_PER_W = B // NW
CHUNK = 128
NCHUNK = B_PER_W // CHUNK
MROWS = 2 * CHUNK
NMAC = B_PER_W // MROWS
NB = 3
LANES = 16
SCALE = math.sqrt(D)


def _emb_body(table_hbm, idx_hbm, out_hbm, idx_v, bufs, gsems, ssems):
    wid = lax.axis_index("s") * NC + lax.axis_index("c")
    base = wid * B_PER_W

    pltpu.sync_copy(idx_hbm.at[wid], idx_v)

    def start_gathers(m, b):
        for h in range(2):
            pltpu.async_copy(
                table_hbm.at[idx_v.at[2 * m + h]],
                bufs[b].at[pl.ds(h * CHUNK, CHUNK)], gsems[b])

    def wait_gathers(m, b):
        for h in range(2):
            pltpu.make_async_copy(
                table_hbm.at[idx_v.at[2 * m + h]],
                bufs[b].at[pl.ds(h * CHUNK, CHUNK)], gsems[b]).wait()

    start_gathers(0, 0)
    start_gathers(1, 1)

    def scale_chunk(buf):
        @plsc.parallel_loop(0, MROWS, step=1, unroll=8)
        def _row(r):
            for c in range(D // LANES):
                sl = pl.ds(c * LANES, LANES)
                buf[r, sl] = buf[r, sl] * SCALE

    def step(it, _):
        m0 = NB * it
        for k in range(NB):
            m = m0 + k
            buf, ssem = bufs[k], ssems[k]
            kw = (k - 1) % NB
            wait_gathers(m, k)
            scale_chunk(buf)
            pltpu.async_copy(
                buf, out_hbm.at[pl.ds(base + m * MROWS, MROWS)], ssem)

            @pl.when(m >= 1)
            def _():
                pltpu.make_async_copy(
                    bufs[kw],
                    out_hbm.at[pl.ds(base + (m - 1) * MROWS, MROWS)],
                    ssems[kw]).wait()

            @pl.when(m < NMAC - 2)
            def _():
                start_gathers(m + 2, kw)
        return 0

    lax.fori_loop(0, NMAC // NB, step, 0)

    m = NMAC - 1
    wait_gathers(m, m % NB)
    scale_chunk(bufs[m % NB])
    pltpu.async_copy(
        bufs[m % NB], out_hbm.at[pl.ds(base + m * MROWS, MROWS)],
        ssems[m % NB])
    pltpu.make_async_copy(
        bufs[(m - 1) % NB],
        out_hbm.at[pl.ds(base + (m - 1) * MROWS, MROWS)],
        ssems[(m - 1) % NB]).wait()
    pltpu.make_async_copy(
        bufs[m % NB], out_hbm.at[pl.ds(base + m * MROWS, MROWS)],
        ssems[m % NB]).wait()


@jax.jit
def kernel(x, table):
    mesh = plsc.VectorSubcoreMesh(core_axis_name="c", subcore_axis_name="s")
    fn = pl.kernel(
        _emb_body,
        out_type=jax.ShapeDtypeStruct((B, D), jnp.float32),
        mesh=mesh,
        scratch_types=[
            pltpu.VMEM((NCHUNK, CHUNK), jnp.int32),
            [pltpu.VMEM((MROWS, D), jnp.float32) for _ in range(NB)],
            [pltpu.SemaphoreType.DMA for _ in range(NB)],
            [pltpu.SemaphoreType.DMA for _ in range(NB)],
        ],
        name="sc_embedding_lookup",
    )
    idx = x.reshape(NW, NCHUNK, CHUNK)
    out = fn(table, idx)
    return out.reshape(BATCH, SEQ, D)


# speedup vs baseline: 1.0044x; 1.0044x over previous
import math

import jax
import jax.numpy as jnp
from jax import lax
from jax.experimental import pallas as pl
from jax.experimental.pallas import tpu as pltpu
from jax.experimental.pallas import tpu_sc as plsc

VOCAB = 1000000
D = 128
BATCH = 4096
SEQ = 200

NC = 2
NS = 16
NW = NC * NS
B = BATCH * SEQ
B_PER_W = B // NW
CHUNK = 128
NCHUNK = B_PER_W // CHUNK
NG = 4
NSB = 2
LANES = 16
SCALE = math.sqrt(D)


def _emb_body(table_hbm, idx_hbm, out_hbm,
              idx_v, gbufs, sbufs, gsems, ssems):
    wid = lax.axis_index("s") * NC + lax.axis_index("c")
    base = wid * B_PER_W

    pltpu.sync_copy(idx_hbm.at[wid], idx_v)

    for g in range(NG):
        pltpu.async_copy(table_hbm.at[idx_v.at[g]], gbufs[g], gsems[g])

    def scale_chunk(src, dst):
        @plsc.parallel_loop(0, CHUNK, step=1, unroll=8)
        def _row(r):
            for c in range(D // LANES):
                sl = pl.ds(c * LANES, LANES)
                dst[r, sl] = src[r, sl] * SCALE

    def step(it, _):
        j0 = NG * it
        for k in range(NG):
            j = j0 + k
            gbuf, gsem = gbufs[k], gsems[k]
            sbuf, ssem = sbufs[k % NSB], ssems[k % NSB]
            pltpu.make_async_copy(table_hbm.at[idx_v.at[j]], gbuf, gsem).wait()

            @pl.when(j >= NSB)
            def _():
                pltpu.make_async_copy(
                    sbuf, out_hbm.at[pl.ds(base + (j - NSB) * CHUNK, CHUNK)],
                    ssem).wait()

            scale_chunk(gbuf, sbuf)
            pltpu.async_copy(
                sbuf, out_hbm.at[pl.ds(base + j * CHUNK, CHUNK)], ssem)

            @pl.when(j < NCHUNK - NG)
            def _():
                pltpu.async_copy(table_hbm.at[idx_v.at[j + NG]], gbuf, gsem)
        return 0

    lax.fori_loop(0, NCHUNK // NG, step, 0)

    for k in range(NSB):
        j = NCHUNK - NSB + k
        pltpu.make_async_copy(
            sbufs[j % NSB], out_hbm.at[pl.ds(base + j * CHUNK, CHUNK)],
            ssems[j % NSB]).wait()


@jax.jit
def kernel(x, table):
    mesh = plsc.VectorSubcoreMesh(core_axis_name="c", subcore_axis_name="s")
    fn = pl.kernel(
        _emb_body,
        out_type=jax.ShapeDtypeStruct((B, D), jnp.float32),
        mesh=mesh,
        scratch_types=[
            pltpu.VMEM((NCHUNK, CHUNK), jnp.int32),
            [pltpu.VMEM((CHUNK, D), jnp.float32) for _ in range(NG)],
            [pltpu.VMEM((CHUNK, D), jnp.float32) for _ in range(NSB)],
            [pltpu.SemaphoreType.DMA for _ in range(NG)],
            [pltpu.SemaphoreType.DMA for _ in range(NSB)],
        ],
        name="sc_embedding_lookup",
    )
    idx = x.reshape(NW, NCHUNK, CHUNK)
    out = fn(table, idx)
    return out.reshape(BATCH, SEQ, D)
